# baseline (speedup 1.0000x reference)
import jax
import jax.numpy as jnp
from jax import lax
from jax.experimental import pallas as pl
from jax.experimental.pallas import tpu as pltpu
from jax.experimental.pallas import tpu_sc as plsc

NUM_UNITS = 1000000
NUM_PHONEMES = 32
BATCH = 16384
HIST = 200

NW = 32
ROWS_PER_W = BATCH // NW
XROWS_PER_GROUP = 4
IDX_PER_GROUP = XROWS_PER_GROUP * HIST
GROUPS_PER_W = ROWS_PER_W // XROWS_PER_GROUP
GATHER_SPLITS = [(0, 128), (128, 128), (256, 128), (384, 128), (512, 128),
                 (640, 128), (768, 32)]
TOTAL = BATCH * HIST


def _gather_kernel(x_hbm, w_hbm, out_hbm, idx_buf, rows, gsem0, gsem1,
                   ssem0, ssem1):
    wid = lax.axis_index("s") * 2 + lax.axis_index("c")
    b0 = wid * ROWS_PER_W
    gsem = (gsem0, gsem1)
    ssem = (ssem0, ssem1)

    def fire_gathers(p):
        for (off, n) in GATHER_SPLITS:
            pltpu.async_copy(w_hbm.at[idx_buf.at[p, pl.ds(off, n)]],
                             rows.at[p, pl.ds(off, n)], gsem[p])

    def drain_gathers(p):
        for (off, n) in GATHER_SPLITS:
            pltpu.make_async_copy(w_hbm.at[idx_buf.at[p, pl.ds(off, n)]],
                                  rows.at[p, pl.ds(off, n)], gsem[p]).wait()

    def fire_stores(p, u):
        for i in range(XROWS_PER_GROUP):
            pltpu.async_copy(rows.at[p, pl.ds(i * HIST, HIST)],
                             out_hbm.at[u * XROWS_PER_GROUP + i], ssem[p])

    def drain_stores(p, u):
        for i in range(XROWS_PER_GROUP):
            pltpu.make_async_copy(rows.at[p, pl.ds(i * HIST, HIST)],
                                  out_hbm.at[u * XROWS_PER_GROUP + i],
                                  ssem[p]).wait()

    def load_idx(p, u):
        pltpu.sync_copy(x_hbm.at[pl.ds(u * IDX_PER_GROUP, IDX_PER_GROUP)],
                        idx_buf.at[p])

    def body(u, p, drain_prev_store, process_prev):
        q = 1 - p
        if drain_prev_store:
            drain_stores(p, u - 2)
        load_idx(p, u)
        fire_gathers(p)
        if process_prev:
            drain_gathers(q)
            fire_stores(q, u - 1)

    u0 = wid * GROUPS_PER_W
    body(u0, 0, False, False)
    body(u0 + 1, 1, False, True)

    def loop_body(k, _):
        u = u0 + 2 * k
        body(u, 0, True, True)
        body(u + 1, 1, True, True)
        return _

    lax.fori_loop(1, GROUPS_PER_W // 2, loop_body, None)

    last = u0 + GROUPS_PER_W - 1
    drain_gathers(1)
    fire_stores(1, last)
    drain_stores(0, last - 1)
    drain_stores(1, last)


@jax.jit
def _run(x_flat, w):
    mesh = plsc.VectorSubcoreMesh(core_axis_name="c", subcore_axis_name="s")
    return pl.kernel(
        _gather_kernel,
        out_type=jax.ShapeDtypeStruct((BATCH, HIST, NUM_PHONEMES),
                                      jnp.float32),
        mesh=mesh,
        scratch_types=[
            pltpu.VMEM((2, IDX_PER_GROUP), jnp.int32),
            pltpu.VMEM((2, IDX_PER_GROUP, NUM_PHONEMES), jnp.float32),
            pltpu.SemaphoreType.DMA,
            pltpu.SemaphoreType.DMA,
            pltpu.SemaphoreType.DMA,
            pltpu.SemaphoreType.DMA,
        ],
        compiler_params=pltpu.CompilerParams(use_tc_tiling_on_sc=False),
    )(x_flat, w)


def kernel(x, W):
    x_flat = x.astype(jnp.int32).reshape(TOTAL)
    return _run(x_flat, W)
